# baseline (speedup 1.0000x reference)
import jax
from jax.experimental import pallas as pl
from jax.experimental.pallas import tpu as pltpu
from jax.experimental.pallas import tpu_sc as plsc


def _rl_body(cu_hbm, out_hbm, cu_s, out_s):
    pltpu.sync_copy(cu_hbm, cu_s)
    for i in range(out_s.shape[0]):
        out_s[i] = cu_s[i + 1] - cu_s[i]
    pltpu.sync_copy(out_s, out_hbm)


def _row_lengths(cu_seqlens):
    n = cu_seqlens.shape[0] - 1
    mesh = plsc.ScalarSubcoreMesh(axis_name="c", num_cores=1)
    return pl.kernel(
        _rl_body,
        out_type=jax.ShapeDtypeStruct((n,), cu_seqlens.dtype),
        mesh=mesh,
        scratch_types=[
            pltpu.SMEM((n + 1,), cu_seqlens.dtype),
            pltpu.SMEM((n,), cu_seqlens.dtype),
        ],
    )(cu_seqlens)


def kernel(values, cu_seqlens):
    return values, _row_lengths(cu_seqlens)
